# baseline (speedup 1.0000x reference)
import functools

import jax
import jax.numpy as jnp
from jax import lax
from jax.experimental import pallas as pl
from jax.experimental.pallas import tpu as pltpu
from jax.experimental.pallas import tpu_sc as plsc

_BATCH = 16384
_LATENT = 32


def _make_gather(batch, latent):
    info = plsc.get_sparse_core_info()
    nw = info.num_cores * info.num_subcores
    assert batch % (8 * nw) == 0
    b_per_w = batch // nw
    grp = 4
    mesh = plsc.VectorSubcoreMesh(core_axis_name="c", subcore_axis_name="s")

    @functools.partial(
        pl.kernel,
        mesh=mesh,
        out_type=jax.ShapeDtypeStruct((batch, 2 * latent), jnp.float32),
        scratch_types=[
            pltpu.VMEM((b_per_w,), jnp.int32),
            pltpu.VMEM((b_per_w,), jnp.int32),
            pltpu.VMEM((grp, latent, 128), jnp.float32),
            pltpu.VMEM((grp, latent, 128), jnp.float32),
            pltpu.VMEM((b_per_w, 2 * latent), jnp.float32),
            pltpu.SemaphoreType.DMA,
            pltpu.SemaphoreType.DMA,
            pltpu.SemaphoreType.DMA,
        ],
        compiler_params=pltpu.CompilerParams(needs_layout_passes=False),
    )
    def gather_kernel(uidx_hbm, iidx_hbm, utab_hbm, itab_hbm, out_hbm,
                      uidx_v, iidx_v, uwin_v, iwin_v, rows_v, isem, gsem,
                      osem):
        wid = lax.axis_index("s") * info.num_cores + lax.axis_index("c")
        base = wid * b_per_w
        ucopy = pltpu.async_copy(uidx_hbm.at[pl.ds(base, b_per_w)], uidx_v, isem)
        icopy = pltpu.async_copy(iidx_hbm.at[pl.ds(base, b_per_w)], iidx_v, isem)
        ucopy.wait()
        icopy.wait()
        f_lo = lax.iota(jnp.int32, 16)
        f_hi = f_lo + 16

        def body(g, _):
            gbase = g * 16
            uvec = uidx_v[pl.ds(gbase, 16)]
            ivec = iidx_v[pl.ds(gbase, 16)]
            ulane = jnp.bitwise_and(uvec, 127)
            ilane = jnp.bitwise_and(ivec, 127)
            for h in range(16 // grp):
                copies = []
                for j in range(grp):
                    e = h * grp + j
                    ub = pl.multiple_of((uvec[e] >> 7) << 7, 128)
                    ib = pl.multiple_of((ivec[e] >> 7) << 7, 128)
                    copies.append(pltpu.async_copy(
                        utab_hbm.at[:, pl.ds(ub, 128)], uwin_v.at[j], gsem))
                    copies.append(pltpu.async_copy(
                        itab_hbm.at[:, pl.ds(ib, 128)], iwin_v.at[j], gsem))
                for c in copies:
                    c.wait()
                for j in range(grp):
                    e = h * grp + j
                    ul = jnp.full((16,), ulane[e], jnp.int32)
                    il = jnp.full((16,), ilane[e], jnp.int32)
                    rows_v[gbase + e, pl.ds(0, 16)] = plsc.load_gather(
                        uwin_v.at[j], [f_lo, ul])
                    rows_v[gbase + e, pl.ds(16, 16)] = plsc.load_gather(
                        uwin_v.at[j], [f_hi, ul])
                    rows_v[gbase + e, pl.ds(32, 16)] = plsc.load_gather(
                        iwin_v.at[j], [f_lo, il])
                    rows_v[gbase + e, pl.ds(48, 16)] = plsc.load_gather(
                        iwin_v.at[j], [f_hi, il])
            return ()

        lax.fori_loop(0, b_per_w // 16, body, ())
        pltpu.async_copy(rows_v, out_hbm.at[pl.ds(base, b_per_w), :], osem).wait()

    return gather_kernel


def kernel(user_idx, item_idx, user_emb, item_emb):
    return _make_gather(_BATCH, _LATENT)(
        user_idx, item_idx, user_emb.T, item_emb.T)
